# baseline (speedup 1.0000x reference)
import functools

import jax
import jax.numpy as jnp
from jax import lax
from jax.experimental import pallas as pl
from jax.experimental.pallas import tpu as pltpu
from jax.experimental.pallas import tpu_sc as plsc

_B = 16384
_D = 40
_NC = 2
_NS = 16
_NW = _NC * _NS
_BPW = _B // _NW
_V = 1000000

_mesh = plsc.VectorSubcoreMesh(core_axis_name="c", subcore_axis_name="s")


@functools.partial(
    pl.kernel,
    out_type=[jax.ShapeDtypeStruct((_B * _D,), jnp.float32)] * 4,
    mesh=_mesh,
    scratch_types=[
        pltpu.VMEM((_BPW,), jnp.int32),
        pltpu.VMEM((_BPW,), jnp.int32),
        pltpu.VMEM((8, _D, 128), jnp.float32),
        pltpu.VMEM((_BPW * _D,), jnp.float32),
        pltpu.VMEM((_BPW * _D,), jnp.float32),
        pltpu.VMEM((_BPW * _D,), jnp.float32),
        pltpu.VMEM((_BPW * _D,), jnp.float32),
        pltpu.SemaphoreType.DMA,
        pltpu.SemaphoreType.DMA,
    ],
    compiler_params=pltpu.CompilerParams(needs_layout_passes=False),
)
def _sc_gather(uid_hbm, iid_hbm, gut, git, mut, mit,
               o0, o1, o2, o3,
               uid_v, iid_v, bufs, f0, f1, f2, f3, sA, so):
    wid = lax.axis_index("s") * _NC + lax.axis_index("c")
    base = wid * _BPW
    pltpu.sync_copy(uid_hbm.at[pl.ds(base, _BPW)], uid_v)
    pltpu.sync_copy(iid_hbm.at[pl.ds(base, _BPW)], iid_v)
    row0 = lax.iota(jnp.int32, 16)
    row1 = row0 + 16
    row2 = row0 + 24

    out_copies = []
    for tab, idx_v, flat, out in (
        (gut, uid_v, f0, o0),
        (git, iid_v, f1, o1),
        (mut, uid_v, f2, o2),
        (mit, iid_v, f3, o3),
    ):
        def body(c, _, tab=tab, idx_v=idx_v, flat=flat):
            j0 = 16 * c
            uv = idx_v[pl.ds(j0, 16)]
            for half in (0, 8):
                hs = []
                for l in range(8):
                    c0 = pl.multiple_of((uv[half + l] >> 7) * 128, 128)
                    hs.append(pltpu.async_copy(
                        tab.at[:, pl.ds(c0, 128)], bufs.at[l], sA))
                for l in range(8):
                    hs[l].wait()
                    col = jnp.full((16,), uv[half + l] & 127, jnp.int32)
                    o = _D * (j0 + half + l)
                    flat[pl.ds(o, 16)] = plsc.load_gather(bufs.at[l], [row0, col])
                    flat[pl.ds(o + 16, 16)] = plsc.load_gather(bufs.at[l], [row1, col])
                    flat[pl.ds(o + 24, 16)] = plsc.load_gather(bufs.at[l], [row2, col])
            return 0

        lax.fori_loop(0, _BPW // 16, body, 0)
        out_copies.append(
            pltpu.async_copy(flat, out.at[pl.ds(base * _D, _BPW * _D)], so))
    for cp in out_copies:
        cp.wait()


_BLK = 2048


def _mlp_body(gu_ref, gi_ref, mu_ref, mi_ref, w1u_ref, w1i_ref, b1_ref,
              w2_ref, b2_ref, wng_ref, wnh_ref, bn_ref, out_ref):
    g = gu_ref[...] * gi_ref[...]
    h1 = jnp.dot(mu_ref[...], w1u_ref[...], preferred_element_type=jnp.float32)
    h1 = h1 + jnp.dot(mi_ref[...], w1i_ref[...], preferred_element_type=jnp.float32)
    h1 = jnp.maximum(h1 + b1_ref[...], 0.0)
    h2 = jnp.dot(h1, w2_ref[...], preferred_element_type=jnp.float32)
    h2 = jnp.maximum(h2 + b2_ref[...], 0.0)
    logit = (jnp.sum(g * wng_ref[...], axis=1, keepdims=True)
             + jnp.sum(h2 * wnh_ref[...], axis=1, keepdims=True)
             + bn_ref[...])
    out_ref[...] = 1.0 / (1.0 + jnp.exp(-logit))


def _mlp_call(gu, gi, mu, mi, w1u, w1i, b1, w2t, b2, wng, wnh, bn):
    grid = (_B // _BLK,)
    row_spec = pl.BlockSpec((_BLK, _D), lambda i: (i, 0))
    full = lambda shape: pl.BlockSpec(shape, lambda i: (0,) * len(shape))
    return pl.pallas_call(
        _mlp_body,
        grid=grid,
        in_specs=[
            row_spec, row_spec, row_spec, row_spec,
            full((_D, 20)), full((_D, 20)), full((1, 20)),
            full((20, 10)), full((1, 10)),
            full((1, _D)), full((1, 10)), full((1, 1)),
        ],
        out_specs=pl.BlockSpec((_BLK, 1), lambda i: (i, 0)),
        out_shape=jax.ShapeDtypeStruct((_B, 1), jnp.float32),
    )(gu, gi, mu, mi, w1u, w1i, b1, w2t, b2, wng, wnh, bn)


def kernel(batch, gmf_user, gmf_item, mlp_user, mlp_item, W1, b1, W2, b2, Wn, bn):
    uid = batch[:, 0]
    iid = batch[:, 1]
    o0, o1, o2, o3 = _sc_gather(uid, iid, gmf_user.T, gmf_item.T,
                                mlp_user.T, mlp_item.T)
    gu = o0.reshape(_B, _D)
    gi = o1.reshape(_B, _D)
    mu = o2.reshape(_B, _D)
    mi = o3.reshape(_B, _D)
    w1u = W1[:, :_D].T
    w1i = W1[:, _D:].T
    w2t = W2.T
    wng = Wn[:, :_D]
    wnh = Wn[:, _D:]
    out = _mlp_call(gu, gi, mu, mi, w1u, w1i, b1.reshape(1, 20), w2t,
                    b2.reshape(1, 10), wng, wnh, bn.reshape(1, 1))
    return out[:, 0]


# speedup vs baseline: 1.0903x; 1.0903x over previous
import functools

import jax
import jax.numpy as jnp
from jax import lax
from jax.experimental import pallas as pl
from jax.experimental.pallas import tpu as pltpu
from jax.experimental.pallas import tpu_sc as plsc

_B = 16384
_D = 40
_NC = 2
_NS = 16
_NW = _NC * _NS
_BPW = _B // _NW
_V = 1000000

_mesh = plsc.VectorSubcoreMesh(core_axis_name="c", subcore_axis_name="s")


@functools.partial(
    pl.kernel,
    out_type=[jax.ShapeDtypeStruct((_B * _D,), jnp.float32)] * 4,
    mesh=_mesh,
    scratch_types=[
        pltpu.VMEM((_BPW,), jnp.int32),
        pltpu.VMEM((_BPW,), jnp.int32),
        pltpu.VMEM((8, _D, 128), jnp.float32),
        pltpu.VMEM((_BPW * _D,), jnp.float32),
        pltpu.VMEM((_BPW * _D,), jnp.float32),
        pltpu.VMEM((_BPW * _D,), jnp.float32),
        pltpu.VMEM((_BPW * _D,), jnp.float32),
        pltpu.SemaphoreType.DMA,
        pltpu.SemaphoreType.DMA,
    ],
    compiler_params=pltpu.CompilerParams(needs_layout_passes=False),
)
def _sc_gather(uid_hbm, iid_hbm, gut, git, mut, mit,
               o0, o1, o2, o3,
               uid_v, iid_v, bufs, f0, f1, f2, f3, sA, so):
    wid = lax.axis_index("s") * _NC + lax.axis_index("c")
    base = wid * _BPW
    pltpu.sync_copy(uid_hbm.at[pl.ds(base, _BPW)], uid_v)
    pltpu.sync_copy(iid_hbm.at[pl.ds(base, _BPW)], iid_v)
    row0 = lax.iota(jnp.int32, 16)
    row1 = row0 + 16
    row2 = row0 + 24

    out_copies = []
    for tab, idx_v, flat, out in (
        (gut, uid_v, f0, o0),
        (git, iid_v, f1, o1),
        (mut, uid_v, f2, o2),
        (mit, iid_v, f3, o3),
    ):
        def body(c, _, tab=tab, idx_v=idx_v, flat=flat):
            j0 = 16 * c
            uv = idx_v[pl.ds(j0, 16)]

            def fire(g):
                hs = []
                for i in range(4):
                    l = 4 * g + i
                    c0 = pl.multiple_of((uv[l] >> 7) * 128, 128)
                    hs.append(pltpu.async_copy(
                        tab.at[:, pl.ds(c0, 128)], bufs.at[4 * (g & 1) + i], sA))
                return hs

            pend = [fire(0), fire(1)]
            for g in range(4):
                for h in pend[g]:
                    h.wait()
                for i in range(4):
                    l = 4 * g + i
                    col = jnp.full((16,), uv[l] & 127, jnp.int32)
                    o = _D * (j0 + l)
                    b = bufs.at[4 * (g & 1) + i]
                    flat[pl.ds(o, 16)] = plsc.load_gather(b, [row0, col])
                    flat[pl.ds(o + 16, 16)] = plsc.load_gather(b, [row1, col])
                    flat[pl.ds(o + 24, 16)] = plsc.load_gather(b, [row2, col])
                if g + 2 < 4:
                    pend.append(fire(g + 2))
            return 0

        lax.fori_loop(0, _BPW // 16, body, 0)
        out_copies.append(
            pltpu.async_copy(flat, out.at[pl.ds(base * _D, _BPW * _D)], so))
    for cp in out_copies:
        cp.wait()


_BLK = 2048


def _mlp_body(gu_ref, gi_ref, mu_ref, mi_ref, w1u_ref, w1i_ref, b1_ref,
              w2_ref, b2_ref, wng_ref, wnh_ref, bn_ref, out_ref):
    g = gu_ref[...] * gi_ref[...]
    h1 = jnp.dot(mu_ref[...], w1u_ref[...], preferred_element_type=jnp.float32)
    h1 = h1 + jnp.dot(mi_ref[...], w1i_ref[...], preferred_element_type=jnp.float32)
    h1 = jnp.maximum(h1 + b1_ref[...], 0.0)
    h2 = jnp.dot(h1, w2_ref[...], preferred_element_type=jnp.float32)
    h2 = jnp.maximum(h2 + b2_ref[...], 0.0)
    logit = (jnp.sum(g * wng_ref[...], axis=1, keepdims=True)
             + jnp.sum(h2 * wnh_ref[...], axis=1, keepdims=True)
             + bn_ref[...])
    out_ref[...] = 1.0 / (1.0 + jnp.exp(-logit))


def _mlp_call(gu, gi, mu, mi, w1u, w1i, b1, w2t, b2, wng, wnh, bn):
    grid = (_B // _BLK,)
    row_spec = pl.BlockSpec((_BLK, _D), lambda i: (i, 0))
    full = lambda shape: pl.BlockSpec(shape, lambda i: (0,) * len(shape))
    return pl.pallas_call(
        _mlp_body,
        grid=grid,
        in_specs=[
            row_spec, row_spec, row_spec, row_spec,
            full((_D, 20)), full((_D, 20)), full((1, 20)),
            full((20, 10)), full((1, 10)),
            full((1, _D)), full((1, 10)), full((1, 1)),
        ],
        out_specs=pl.BlockSpec((_BLK, 1), lambda i: (i, 0)),
        out_shape=jax.ShapeDtypeStruct((_B, 1), jnp.float32),
    )(gu, gi, mu, mi, w1u, w1i, b1, w2t, b2, wng, wnh, bn)


def kernel(batch, gmf_user, gmf_item, mlp_user, mlp_item, W1, b1, W2, b2, Wn, bn):
    uid = batch[:, 0]
    iid = batch[:, 1]
    o0, o1, o2, o3 = _sc_gather(uid, iid, gmf_user.T, gmf_item.T,
                                mlp_user.T, mlp_item.T)
    gu = o0.reshape(_B, _D)
    gi = o1.reshape(_B, _D)
    mu = o2.reshape(_B, _D)
    mi = o3.reshape(_B, _D)
    w1u = W1[:, :_D].T
    w1i = W1[:, _D:].T
    w2t = W2.T
    wng = Wn[:, :_D]
    wnh = Wn[:, _D:]
    out = _mlp_call(gu, gi, mu, mi, w1u, w1i, b1.reshape(1, 20), w2t,
                    b2.reshape(1, 10), wng, wnh, bn.reshape(1, 1))
    return out[:, 0]
